# baseline (speedup 1.0000x reference)
import jax
import jax.numpy as jnp
from jax.experimental import pallas as pl
from jax.experimental.pallas import tpu as pltpu

_T = 32768
_C = 256
_CHUNK = 1024
_NBUF = 12

_SCHEDULE = []
_sizes = [256, 256, 512] + [1024] * 30 + [512, 256, 256]
assert sum(_sizes) == _T
_off = 0
for _s in _sizes:
    _SCHEDULE.append((_off, _s))
    _off += _s


def _agg_kernel(x1_hbm, x2_hbm, w_ref, b_ref, o_hbm,
                x1_buf, x2_buf, o_buf, in_sems, out_sems):
    def in_copies(idx, slot):
        base, rows = _SCHEDULE[idx]
        c1 = pltpu.make_async_copy(
            x1_hbm.at[pl.ds(base, rows)], x1_buf.at[slot, pl.ds(0, rows)],
            in_sems.at[slot, 0])
        c2 = pltpu.make_async_copy(
            x2_hbm.at[pl.ds(base, rows)], x2_buf.at[slot, pl.ds(0, rows)],
            in_sems.at[slot, 1])
        return c1, c2

    def out_copy(idx, slot):
        base, rows = _SCHEDULE[idx]
        return pltpu.make_async_copy(
            o_buf.at[slot, pl.ds(0, rows)], o_hbm.at[pl.ds(base, rows)],
            out_sems.at[slot])

    for idx in range(_NBUF):
        for c in in_copies(idx, idx):
            c.start()

    w1 = w_ref[:_C, :]
    w2 = w_ref[_C:, :]
    bias = b_ref[...].reshape(1, _C)

    n = len(_SCHEDULE)
    for idx in range(n):
        slot = idx % _NBUF
        rows = _SCHEDULE[idx][1]
        c1, c2 = in_copies(idx, slot)
        c1.wait()
        c2.wait()
        if idx >= _NBUF:
            out_copy(idx - _NBUF, slot).wait()
        acc = jnp.dot(x1_buf[slot, :rows], w1, preferred_element_type=jnp.float32)
        acc = acc + jnp.dot(x2_buf[slot, :rows], w2,
                            preferred_element_type=jnp.float32)
        o_buf[slot, :rows] = acc + bias
        out_copy(idx, slot).start()
        nxt = idx + _NBUF
        if nxt < n:
            for c in in_copies(nxt, slot):
                c.start()

    for idx in range(n - _NBUF, n):
        out_copy(idx, idx % _NBUF).wait()


def kernel(x_1, x_2, W, b):
    return pl.pallas_call(
        _agg_kernel,
        in_specs=[
            pl.BlockSpec(memory_space=pl.ANY),
            pl.BlockSpec(memory_space=pl.ANY),
            pl.BlockSpec(memory_space=pltpu.VMEM),
            pl.BlockSpec(memory_space=pltpu.VMEM),
        ],
        out_specs=pl.BlockSpec(memory_space=pl.ANY),
        out_shape=jax.ShapeDtypeStruct((_T, _C), jnp.float32),
        scratch_shapes=[
            pltpu.VMEM((_NBUF, _CHUNK, _C), jnp.float32),
            pltpu.VMEM((_NBUF, _CHUNK, _C), jnp.float32),
            pltpu.VMEM((_NBUF, _CHUNK, _C), jnp.float32),
            pltpu.SemaphoreType.DMA((_NBUF, 2)),
            pltpu.SemaphoreType.DMA((_NBUF,)),
        ],
    )(x_1, x_2, W, b)


# speedup vs baseline: 1.0065x; 1.0065x over previous
import jax
import jax.numpy as jnp
from jax.experimental import pallas as pl
from jax.experimental.pallas import tpu as pltpu

_T = 32768
_C = 256
_CHUNK = 2048
_NBUF = 6

_SCHEDULE = []
_sizes = [256, 256, 512] + [2048] * 15 + [512, 256, 256]
assert sum(_sizes) == _T
_off = 0
for _s in _sizes:
    _SCHEDULE.append((_off, _s))
    _off += _s


def _agg_kernel(x1_hbm, x2_hbm, w_ref, b_ref, o_hbm,
                x1_buf, x2_buf, o_buf, in_sems, out_sems):
    def in_copies(idx, slot):
        base, rows = _SCHEDULE[idx]
        c1 = pltpu.make_async_copy(
            x1_hbm.at[pl.ds(base, rows)], x1_buf.at[slot, pl.ds(0, rows)],
            in_sems.at[slot, 0])
        c2 = pltpu.make_async_copy(
            x2_hbm.at[pl.ds(base, rows)], x2_buf.at[slot, pl.ds(0, rows)],
            in_sems.at[slot, 1])
        return c1, c2

    def out_copy(idx, slot):
        base, rows = _SCHEDULE[idx]
        return pltpu.make_async_copy(
            o_buf.at[slot, pl.ds(0, rows)], o_hbm.at[pl.ds(base, rows)],
            out_sems.at[slot])

    for idx in range(_NBUF):
        for c in in_copies(idx, idx):
            c.start()

    w1 = w_ref[:_C, :]
    w2 = w_ref[_C:, :]
    bias = b_ref[...].reshape(1, _C)

    n = len(_SCHEDULE)
    for idx in range(n):
        slot = idx % _NBUF
        rows = _SCHEDULE[idx][1]
        c1, c2 = in_copies(idx, slot)
        c1.wait()
        c2.wait()
        if idx >= _NBUF:
            out_copy(idx - _NBUF, slot).wait()
        acc = jnp.dot(x1_buf[slot, :rows], w1, preferred_element_type=jnp.float32)
        acc = acc + jnp.dot(x2_buf[slot, :rows], w2,
                            preferred_element_type=jnp.float32)
        o_buf[slot, :rows] = acc + bias
        out_copy(idx, slot).start()
        nxt = idx + _NBUF
        if nxt < n:
            for c in in_copies(nxt, slot):
                c.start()

    for idx in range(n - _NBUF, n):
        out_copy(idx, idx % _NBUF).wait()


def kernel(x_1, x_2, W, b):
    return pl.pallas_call(
        _agg_kernel,
        in_specs=[
            pl.BlockSpec(memory_space=pl.ANY),
            pl.BlockSpec(memory_space=pl.ANY),
            pl.BlockSpec(memory_space=pltpu.VMEM),
            pl.BlockSpec(memory_space=pltpu.VMEM),
        ],
        out_specs=pl.BlockSpec(memory_space=pl.ANY),
        out_shape=jax.ShapeDtypeStruct((_T, _C), jnp.float32),
        scratch_shapes=[
            pltpu.VMEM((_NBUF, _CHUNK, _C), jnp.float32),
            pltpu.VMEM((_NBUF, _CHUNK, _C), jnp.float32),
            pltpu.VMEM((_NBUF, _CHUNK, _C), jnp.float32),
            pltpu.SemaphoreType.DMA((_NBUF, 2)),
            pltpu.SemaphoreType.DMA((_NBUF,)),
        ],
    )(x_1, x_2, W, b)
